# baseline (speedup 1.0000x reference)
import functools

import jax
import jax.numpy as jnp
from jax import lax
from jax.experimental import pallas as pl
from jax.experimental.pallas import tpu as pltpu
from jax.experimental.pallas import tpu_sc as plsc

L = 16


def _native_1d(x2d):
    R, C = x2d.shape
    z = x2d.reshape(R // 8, 8, C // 128, 128).transpose(0, 2, 1, 3)
    return z.reshape(R * C)


@functools.lru_cache(maxsize=None)
def _build(U, I, K, B):
    NC, NS = 2, 16
    NW = NC * NS
    assert B % (NW * L) == 0 and U % 8 == 0 and I % 128 == 0
    PB = B // NW
    PCH = PB // L
    NE = PB * K
    ECH = NE // L
    IDX_MINOR = 128
    CPR = IDX_MINOR // L
    assert NE % IDX_MINOR == 0
    IDX_MAJOR = NE // IDX_MINOR
    HALF = (IDX_MAJOR + 1) // 2
    ROWS8 = 8 * I

    mesh = plsc.VectorSubcoreMesh(core_axis_name="c", subcore_axis_name="s")

    @functools.partial(
        pl.kernel,
        out_type=jax.ShapeDtypeStruct((B,), jnp.float32),
        mesh=mesh,
        compiler_params=pltpu.CompilerParams(needs_layout_passes=False),
        scratch_types=[
            pltpu.VMEM((PB,), jnp.int32),
            pltpu.VMEM((PB,), jnp.int32),
            pltpu.VMEM((PB,), jnp.int32),
            pltpu.VMEM((PB,), jnp.int32),
            pltpu.VMEM((PB,), jnp.int32),
            pltpu.VMEM((PB,), jnp.float32),
            pltpu.VMEM((PB,), jnp.float32),
            pltpu.VMEM((I,), jnp.float32),
            pltpu.VMEM((U,), jnp.float32),
            pltpu.VMEM((IDX_MAJOR, IDX_MINOR), jnp.int32),
            pltpu.VMEM((IDX_MAJOR, IDX_MINOR), jnp.int32),
            pltpu.VMEM((IDX_MAJOR, IDX_MINOR), jnp.int32),
            pltpu.VMEM((IDX_MAJOR, IDX_MINOR), jnp.int32),
            pltpu.VMEM((IDX_MAJOR, IDX_MINOR), jnp.int32),
            pltpu.VMEM((IDX_MAJOR, IDX_MINOR), jnp.int32),
            pltpu.VMEM((IDX_MAJOR, IDX_MINOR), jnp.float32),
            pltpu.VMEM((IDX_MAJOR, IDX_MINOR), jnp.float32),
            pltpu.VMEM((IDX_MAJOR, IDX_MINOR), jnp.float32),
            pltpu.VMEM((K, PB), jnp.float32),
            pltpu.VMEM((PB,), jnp.float32),
            pltpu.SemaphoreType.DMA,
            pltpu.SemaphoreType.DMA,
            pltpu.SemaphoreType.DMA,
            pltpu.SemaphoreType.DMA,
            pltpu.SemaphoreType.DMA,
            pltpu.SemaphoreType.DMA,
        ],
    )
    def launch(uidx_hbm, tidx_hbm, qn_hbm, rn_hbm, wn_hbm, buser_hbm,
               bitem_hbm, out_hbm,
               u_v, t_v, basep_v, upart_v, tpart_v, bu_v, bi_v, bitem_v,
               buser_v, jv_v, kv_v, qidx_v, q_v, widx_v, ridx_v, w_v, r_v,
               adj_v, t_prod, out_v, semq0, semq1, semw0, semw1, semr0,
               semr1):
        cid = lax.axis_index("c")
        sid = lax.axis_index("s")
        wid = sid * NC + cid
        base = wid * PB

        pltpu.sync_copy(uidx_hbm.at[pl.ds(base, PB)], u_v)
        pltpu.sync_copy(tidx_hbm.at[pl.ds(base, PB)], t_v)
        pltpu.sync_copy(buser_hbm, buser_v)
        pltpu.sync_copy(bitem_hbm, bitem_v)

        lanes = lax.iota(jnp.int32, L)
        halves = [(0, HALF, semq0, semw0, semr0),
                  (HALF, IDX_MAJOR, semq1, semw1, semr1)]

        @plsc.parallel_loop(0, PCH, unroll=2)
        def p1(i):
            sl = pl.ds(i * L, L)
            uvec = u_v[sl]
            tvec = t_v[sl]
            up = (uvec >> 3) * ROWS8 + (uvec & 7) * 128
            tp = (tvec >> 7) * 1024 + (tvec & 127)
            upart_v[sl] = up
            tpart_v[sl] = tp
            basep_v[sl] = up + tp
            bu_v[sl] = plsc.load_gather(buser_v, [uvec])
            bi_v[sl] = plsc.load_gather(bitem_v, [tvec])

        for m0, m1, semq, _, _ in halves:
            @plsc.parallel_loop(m0, m1)
            def p2(m, _semq=semq):
                for cc in range(CPR):
                    sl = pl.ds(cc * L, L)
                    n_v = (m * CPR + cc) * L + lanes
                    jv = n_v // K
                    kv = n_v - jv * K
                    jv_v[m, sl] = jv
                    kv_v[m, sl] = kv
                    bp = plsc.load_gather(basep_v, [jv])
                    qidx_v[m, sl] = kv * (U * I) + bp
                pltpu.async_copy(qn_hbm.at[qidx_v.at[m]], q_v.at[m], _semq)

        qdummy = qn_hbm.at[pl.ds(0, IDX_MINOR)]
        fdummy = wn_hbm.at[pl.ds(0, IDX_MINOR)]

        for m0, m1, semq, semw, semr in halves:
            for m in range(m0, m1):
                pltpu.make_async_copy(qdummy, q_v.at[m], semq).wait()

            @plsc.parallel_loop(m0, m1, unroll=2)
            def p3(m, _semw=semw, _semr=semr):
                for cc in range(CPR):
                    sl = pl.ds(cc * L, L)
                    jv = jv_v[m, sl]
                    qv = q_v[m, sl]
                    tp = plsc.load_gather(tpart_v, [jv])
                    up = plsc.load_gather(upart_v, [jv])
                    bu_b = plsc.load_gather(bu_v, [jv])
                    bj = plsc.load_gather(bitem_v, [qv])
                    widx_v[m, sl] = (qv >> 3) * ROWS8 + (qv & 7) * 128 + tp
                    ridx_v[m, sl] = up + (qv >> 7) * 1024 + (qv & 127)
                    adj_v[m, sl] = bu_b + bj
                pltpu.async_copy(wn_hbm.at[widx_v.at[m]], w_v.at[m], _semw)
                pltpu.async_copy(rn_hbm.at[ridx_v.at[m]], r_v.at[m], _semr)

        for m0, m1, _, semw, semr in halves:
            for m in range(m0, m1):
                pltpu.make_async_copy(fdummy, w_v.at[m], semw).wait()
                pltpu.make_async_copy(fdummy, r_v.at[m], semr).wait()

            @plsc.parallel_loop(m0 * CPR, m1 * CPR, unroll=4)
            def p5(c):
                maj = c // CPR
                sl = pl.ds((c - maj * CPR) * L, L)
                prod = w_v[maj, sl] * (r_v[maj, sl] - adj_v[maj, sl])
                plsc.store_scatter(
                    t_prod, [kv_v[maj, sl], jv_v[maj, sl]], prod)

        @plsc.parallel_loop(0, PCH, unroll=2)
        def p6(o):
            sl = pl.ds(o * L, L)
            acc = t_prod[0, sl]
            for e in range(1, K):
                acc = acc + t_prod[e, sl]
            out_v[sl] = bu_v[sl] + bi_v[sl] + acc * (1.0 / K)

        pltpu.sync_copy(out_v, out_hbm.at[pl.ds(base, PB)])

    return launch


def kernel(user_idx, item_idx, qtus, rating_matrix, weight, b_user, b_item):
    U, I = rating_matrix.shape
    K = qtus.shape[-1]
    B = user_idx.shape[0]
    launch = _build(U, I, K, B)
    q_native = _native_1d(qtus.transpose(2, 0, 1).reshape(K * U, I))
    return launch(
        user_idx.astype(jnp.int32),
        item_idx.astype(jnp.int32),
        q_native,
        _native_1d(rating_matrix),
        _native_1d(weight),
        b_user,
        b_item,
    )


# speedup vs baseline: 1.0217x; 1.0217x over previous
import functools

import jax
import jax.numpy as jnp
from jax import lax
from jax.experimental import pallas as pl
from jax.experimental.pallas import tpu as pltpu
from jax.experimental.pallas import tpu_sc as plsc

L = 16


def _native_1d(x2d):
    R, C = x2d.shape
    z = x2d.reshape(R // 8, 8, C // 128, 128).transpose(0, 2, 1, 3)
    return z.reshape(R * C)


@functools.lru_cache(maxsize=None)
def _build(U, I, K, B):
    NC, NS = 2, 16
    NW = NC * NS
    assert B % (NW * L) == 0 and U % 8 == 0 and I % 128 == 0
    PB = B // NW
    PCH = PB // L
    NE = PB * K
    ECH = NE // L
    IDX_MINOR = 128
    CPR = IDX_MINOR // L
    assert NE % IDX_MINOR == 0
    IDX_MAJOR = NE // IDX_MINOR
    NSEG = 5
    SEG = IDX_MAJOR // NSEG
    assert SEG * NSEG == IDX_MAJOR
    ROWS8 = 8 * I

    mesh = plsc.VectorSubcoreMesh(core_axis_name="c", subcore_axis_name="s")

    @functools.partial(
        pl.kernel,
        out_type=jax.ShapeDtypeStruct((B,), jnp.float32),
        mesh=mesh,
        compiler_params=pltpu.CompilerParams(needs_layout_passes=False),
        scratch_types=[
            pltpu.VMEM((PB,), jnp.int32),
            pltpu.VMEM((PB,), jnp.int32),
            pltpu.VMEM((PB,), jnp.int32),
            pltpu.VMEM((PB,), jnp.int32),
            pltpu.VMEM((PB,), jnp.int32),
            pltpu.VMEM((PB,), jnp.float32),
            pltpu.VMEM((PB,), jnp.float32),
            pltpu.VMEM((I,), jnp.float32),
            pltpu.VMEM((U,), jnp.float32),
            pltpu.VMEM((IDX_MAJOR, IDX_MINOR), jnp.int32),
            pltpu.VMEM((IDX_MAJOR, IDX_MINOR), jnp.int32),
            pltpu.VMEM((IDX_MAJOR, IDX_MINOR), jnp.int32),
            pltpu.VMEM((IDX_MAJOR, IDX_MINOR), jnp.int32),
            pltpu.VMEM((IDX_MAJOR, IDX_MINOR), jnp.int32),
            pltpu.VMEM((IDX_MAJOR, IDX_MINOR), jnp.int32),
            pltpu.VMEM((IDX_MAJOR, IDX_MINOR), jnp.float32),
            pltpu.VMEM((IDX_MAJOR, IDX_MINOR), jnp.float32),
            pltpu.VMEM((IDX_MAJOR, IDX_MINOR), jnp.float32),
            pltpu.VMEM((K, PB), jnp.float32),
            pltpu.VMEM((PB,), jnp.float32),
        ] + [pltpu.SemaphoreType.DMA] * (3 * NSEG),
    )
    def launch(uidx_hbm, tidx_hbm, qn_hbm, rn_hbm, wn_hbm, buser_hbm,
               bitem_hbm, out_hbm,
               u_v, t_v, basep_v, upart_v, tpart_v, bu_v, bi_v, bitem_v,
               buser_v, jv_v, kv_v, qidx_v, q_v, widx_v, ridx_v, w_v, r_v,
               adj_v, t_prod, out_v, *sems):
        semqs = sems[0:NSEG]
        semws = sems[NSEG:2 * NSEG]
        semrs = sems[2 * NSEG:3 * NSEG]
        cid = lax.axis_index("c")
        sid = lax.axis_index("s")
        wid = sid * NC + cid
        base = wid * PB

        pltpu.sync_copy(uidx_hbm.at[pl.ds(base, PB)], u_v)
        pltpu.sync_copy(tidx_hbm.at[pl.ds(base, PB)], t_v)
        pltpu.sync_copy(buser_hbm, buser_v)
        pltpu.sync_copy(bitem_hbm, bitem_v)

        lanes = lax.iota(jnp.int32, L)
        halves = [(h * SEG, (h + 1) * SEG, semqs[h], semws[h], semrs[h])
                  for h in range(NSEG)]

        @plsc.parallel_loop(0, ECH, unroll=4)
        def p0(c):
            n_v = c * L + lanes
            jv = n_v // K
            maj = c // CPR
            sl = pl.ds((c - maj * CPR) * L, L)
            jv_v[maj, sl] = jv
            kv_v[maj, sl] = n_v - jv * K

        @plsc.parallel_loop(0, PCH, unroll=2)
        def p1(i):
            sl = pl.ds(i * L, L)
            uvec = u_v[sl]
            tvec = t_v[sl]
            up = (uvec >> 3) * ROWS8 + (uvec & 7) * 128
            tp = (tvec >> 7) * 1024 + (tvec & 127)
            upart_v[sl] = up
            tpart_v[sl] = tp
            basep_v[sl] = up + tp
            bu_v[sl] = plsc.load_gather(buser_v, [uvec])
            bi_v[sl] = plsc.load_gather(bitem_v, [tvec])

        for m0, m1, semq, _, _ in halves:
            @plsc.parallel_loop(m0, m1)
            def p2(m, _semq=semq):
                for cc in range(CPR):
                    sl = pl.ds(cc * L, L)
                    bp = plsc.load_gather(basep_v, [jv_v[m, sl]])
                    qidx_v[m, sl] = kv_v[m, sl] * (U * I) + bp
                pltpu.async_copy(qn_hbm.at[qidx_v.at[m]], q_v.at[m], _semq)

        qdummy = qn_hbm.at[pl.ds(0, IDX_MINOR)]
        fdummy = wn_hbm.at[pl.ds(0, IDX_MINOR)]

        for m0, m1, semq, semw, semr in halves:
            for m in range(m0, m1):
                pltpu.make_async_copy(qdummy, q_v.at[m], semq).wait()

            @plsc.parallel_loop(m0, m1)
            def p3(m, _semw=semw, _semr=semr):
                for cc in range(CPR):
                    sl = pl.ds(cc * L, L)
                    jv = jv_v[m, sl]
                    qv = q_v[m, sl]
                    tp = plsc.load_gather(tpart_v, [jv])
                    up = plsc.load_gather(upart_v, [jv])
                    bu_b = plsc.load_gather(bu_v, [jv])
                    bj = plsc.load_gather(bitem_v, [qv])
                    widx_v[m, sl] = (qv >> 3) * ROWS8 + (qv & 7) * 128 + tp
                    ridx_v[m, sl] = up + (qv >> 7) * 1024 + (qv & 127)
                    adj_v[m, sl] = bu_b + bj
                pltpu.async_copy(wn_hbm.at[widx_v.at[m]], w_v.at[m], _semw)
                pltpu.async_copy(rn_hbm.at[ridx_v.at[m]], r_v.at[m], _semr)

        for m0, m1, _, semw, semr in halves:
            for m in range(m0, m1):
                pltpu.make_async_copy(fdummy, w_v.at[m], semw).wait()
                pltpu.make_async_copy(fdummy, r_v.at[m], semr).wait()

            @plsc.parallel_loop(m0 * CPR, m1 * CPR, unroll=2)
            def p5(c):
                maj = c // CPR
                sl = pl.ds((c - maj * CPR) * L, L)
                prod = w_v[maj, sl] * (r_v[maj, sl] - adj_v[maj, sl])
                plsc.store_scatter(
                    t_prod, [kv_v[maj, sl], jv_v[maj, sl]], prod)

        @plsc.parallel_loop(0, PCH)
        def p6(o):
            sl = pl.ds(o * L, L)
            acc = t_prod[0, sl]
            for e in range(1, K):
                acc = acc + t_prod[e, sl]
            out_v[sl] = bu_v[sl] + bi_v[sl] + acc * (1.0 / K)

        pltpu.sync_copy(out_v, out_hbm.at[pl.ds(base, PB)])

    return launch


def kernel(user_idx, item_idx, qtus, rating_matrix, weight, b_user, b_item):
    U, I = rating_matrix.shape
    K = qtus.shape[-1]
    B = user_idx.shape[0]
    launch = _build(U, I, K, B)
    q_native = _native_1d(qtus.transpose(2, 0, 1).reshape(K * U, I))
    return launch(
        user_idx.astype(jnp.int32),
        item_idx.astype(jnp.int32),
        q_native,
        _native_1d(rating_matrix),
        _native_1d(weight),
        b_user,
        b_item,
    )


# speedup vs baseline: 1.1320x; 1.1080x over previous
import functools

import jax
import jax.numpy as jnp
from jax import lax
from jax.experimental import pallas as pl
from jax.experimental.pallas import tpu as pltpu
from jax.experimental.pallas import tpu_sc as plsc

L = 16


def _native_1d(x2d):
    R, C = x2d.shape
    z = x2d.reshape(R // 8, 8, C // 128, 128).transpose(0, 2, 1, 3)
    return z.reshape(R * C)


@functools.lru_cache(maxsize=None)
def _build(U, I, K, B):
    NC, NS = 2, 16
    NW = NC * NS
    assert B % (NW * L) == 0 and U % 8 == 0 and I % 128 == 0
    PB = B // NW
    PCH = PB // L
    NE = PB * K
    ECH = NE // L
    IDX_MINOR = 128
    CPR = IDX_MINOR // L
    assert NE % IDX_MINOR == 0
    IDX_MAJOR = NE // IDX_MINOR
    NSEG = 5
    SEG = IDX_MAJOR // NSEG
    assert SEG * NSEG == IDX_MAJOR
    ROWS8 = 8 * I

    mesh = plsc.VectorSubcoreMesh(core_axis_name="c", subcore_axis_name="s")

    @functools.partial(
        pl.kernel,
        out_type=jax.ShapeDtypeStruct((B,), jnp.float32),
        mesh=mesh,
        compiler_params=pltpu.CompilerParams(needs_layout_passes=False),
        scratch_types=[
            pltpu.VMEM((PB,), jnp.int32),
            pltpu.VMEM((PB,), jnp.int32),
            pltpu.VMEM((PB,), jnp.int32),
            pltpu.VMEM((PB,), jnp.int32),
            pltpu.VMEM((PB,), jnp.int32),
            pltpu.VMEM((PB,), jnp.float32),
            pltpu.VMEM((PB,), jnp.float32),
            pltpu.VMEM((I,), jnp.float32),
            pltpu.VMEM((U,), jnp.float32),
            pltpu.VMEM((IDX_MAJOR, IDX_MINOR), jnp.int32),
            pltpu.VMEM((IDX_MAJOR, IDX_MINOR), jnp.int32),
            pltpu.VMEM((IDX_MAJOR, IDX_MINOR), jnp.int32),
            pltpu.VMEM((IDX_MAJOR, IDX_MINOR), jnp.int32),
            pltpu.VMEM((IDX_MAJOR, IDX_MINOR), jnp.int32),
            pltpu.VMEM((IDX_MAJOR, IDX_MINOR), jnp.float32),
            pltpu.VMEM((IDX_MAJOR, IDX_MINOR), jnp.float32),
            pltpu.VMEM_SHARED((I * I,), jnp.float32),
            pltpu.VMEM((K, PB), jnp.float32),
            pltpu.VMEM((PB,), jnp.float32),
        ] + [pltpu.SemaphoreType.DMA] * (3 * NSEG + 1),
    )
    def launch(uidx_hbm, tidx_hbm, qn_hbm, rn_hbm, wn_hbm, buser_hbm,
               bitem_hbm, out_hbm,
               u_v, t_v, basep_v, upart_v, tpart_v, bu_v, bi_v, bitem_v,
               buser_v, jv_v, kv_v, q_v, widx_v, ridx_v, w_v, r_v,
               w_sh, t_prod, out_v, *sems):
        qidx_v = widx_v
        semst = sems[3 * NSEG]
        semqs = sems[0:NSEG]
        semws = sems[NSEG:2 * NSEG]
        semrs = sems[2 * NSEG:3 * NSEG]
        cid = lax.axis_index("c")
        sid = lax.axis_index("s")
        wid = sid * NC + cid
        base = wid * PB

        pltpu.sync_copy(uidx_hbm.at[pl.ds(base, PB)], u_v)
        pltpu.sync_copy(tidx_hbm.at[pl.ds(base, PB)], t_v)
        pltpu.sync_copy(buser_hbm, buser_v)
        pltpu.sync_copy(bitem_hbm, bitem_v)

        wpart = (I * I) // NS
        stage = pltpu.async_copy(
            wn_hbm.at[pl.ds(sid * wpart, wpart)],
            w_sh.at[pl.ds(sid * wpart, wpart)], semst)

        lanes = lax.iota(jnp.int32, L)
        halves = [(h * SEG, (h + 1) * SEG, semqs[h], semws[h], semrs[h])
                  for h in range(NSEG)]

        @plsc.parallel_loop(0, ECH, unroll=4)
        def p0(c):
            n_v = c * L + lanes
            jv = n_v // K
            maj = c // CPR
            sl = pl.ds((c - maj * CPR) * L, L)
            jv_v[maj, sl] = jv
            kv_v[maj, sl] = n_v - jv * K

        @plsc.parallel_loop(0, PCH, unroll=2)
        def p1(i):
            sl = pl.ds(i * L, L)
            uvec = u_v[sl]
            tvec = t_v[sl]
            up = (uvec >> 3) * ROWS8 + (uvec & 7) * 128
            tp = (tvec >> 7) * 1024 + (tvec & 127)
            upart_v[sl] = up
            tpart_v[sl] = tp
            basep_v[sl] = up + tp
            bu_v[sl] = plsc.load_gather(buser_v, [uvec])
            bi_v[sl] = plsc.load_gather(bitem_v, [tvec])

        for m0, m1, semq, _, _ in halves:
            @plsc.parallel_loop(m0, m1)
            def p2(m, _semq=semq):
                for cc in range(CPR):
                    sl = pl.ds(cc * L, L)
                    bp = plsc.load_gather(basep_v, [jv_v[m, sl]])
                    qidx_v[m, sl] = kv_v[m, sl] * (U * I) + bp
                pltpu.async_copy(qn_hbm.at[qidx_v.at[m]], q_v.at[m], _semq)

        qdummy = qn_hbm.at[pl.ds(0, IDX_MINOR)]
        fdummy = wn_hbm.at[pl.ds(0, IDX_MINOR)]

        stage.wait()
        plsc.subcore_barrier()

        for m0, m1, semq, semw, semr in halves:
            for m in range(m0, m1):
                pltpu.make_async_copy(qdummy, q_v.at[m], semq).wait()

            @plsc.parallel_loop(m0, m1)
            def p3(m, _semw=semw, _semr=semr):
                for cc in range(CPR):
                    sl = pl.ds(cc * L, L)
                    jv = jv_v[m, sl]
                    qv = q_v[m, sl]
                    tp = plsc.load_gather(tpart_v, [jv])
                    up = plsc.load_gather(upart_v, [jv])
                    bu_b = plsc.load_gather(bu_v, [jv])
                    bj = plsc.load_gather(bitem_v, [qv])
                    widx_v[m, sl] = (qv >> 3) * ROWS8 + (qv & 7) * 128 + tp
                    ridx_v[m, sl] = up + (qv >> 7) * 1024 + (qv & 127)
                    q_v[m, sl] = plsc.bitcast(bu_b + bj, jnp.int32)
                pltpu.async_copy(w_sh.at[widx_v.at[m]], w_v.at[m], _semw)
                pltpu.async_copy(rn_hbm.at[ridx_v.at[m]], r_v.at[m], _semr)

        for m0, m1, _, semw, semr in halves:
            for m in range(m0, m1):
                pltpu.make_async_copy(fdummy, w_v.at[m], semw).wait()
                pltpu.make_async_copy(fdummy, r_v.at[m], semr).wait()

            @plsc.parallel_loop(m0 * CPR, m1 * CPR, unroll=2)
            def p5(c):
                maj = c // CPR
                sl = pl.ds((c - maj * CPR) * L, L)
                adj = plsc.bitcast(q_v[maj, sl], jnp.float32)
                prod = w_v[maj, sl] * (r_v[maj, sl] - adj)
                plsc.store_scatter(
                    t_prod, [kv_v[maj, sl], jv_v[maj, sl]], prod)

        @plsc.parallel_loop(0, PCH)
        def p6(o):
            sl = pl.ds(o * L, L)
            acc = t_prod[0, sl]
            for e in range(1, K):
                acc = acc + t_prod[e, sl]
            out_v[sl] = bu_v[sl] + bi_v[sl] + acc * (1.0 / K)

        pltpu.sync_copy(out_v, out_hbm.at[pl.ds(base, PB)])

    return launch


def kernel(user_idx, item_idx, qtus, rating_matrix, weight, b_user, b_item):
    U, I = rating_matrix.shape
    K = qtus.shape[-1]
    B = user_idx.shape[0]
    launch = _build(U, I, K, B)
    q_native = _native_1d(qtus.transpose(2, 0, 1).reshape(K * U, I))
    return launch(
        user_idx.astype(jnp.int32),
        item_idx.astype(jnp.int32),
        q_native,
        _native_1d(rating_matrix),
        _native_1d(weight),
        b_user,
        b_item,
    )
